# Initial kernel scaffold; baseline (speedup 1.0000x reference)
import jax
import jax.numpy as jnp
from jax.experimental import pallas as pl


def kernel(x, d, edge_index, W_theta, b_theta, W_phi, b_phi):
    raise NotImplementedError("write your pallas kernel here")


# speedup vs baseline: 1.3243x; 1.3243x over previous
import functools

import jax
import jax.numpy as jnp
from jax import lax
from jax.experimental import pallas as pl
from jax.experimental.pallas import tpu as pltpu, tpu_sc as plsc

N = 10000
E = 320000
D = 128
NW = 32
R = 320
NP = NW * R
CHUNK = 8000
NG = CHUNK // 16
NCHUNKS = E // CHUNK
CAP = 128
FLUSH_AT = 96
NEG = float("-inf")


def _dg(v, idx):
    dn = lax.GatherDimensionNumbers(
        offset_dims=(), collapsed_slice_dims=(0,), start_index_map=(0,))
    return lax.gather(v, idx[:, None], dn, slice_sizes=(1,),
                      mode=lax.GatherScatterMode.PROMISE_IN_BOUNDS)


def _mm_body(x_ref, wt_ref, wp_ref, b2_ref, xt_ref, xp_ref):
    xx = x_ref[...]
    dn = (((1,), (1,)), ((), ()))
    xt_ref[...] = lax.dot_general(xx, wt_ref[...], dn,
                                  preferred_element_type=jnp.float32)
    xp_ref[...] = lax.dot_general(xx, wp_ref[...], dn,
                                  preferred_element_type=jnp.float32) + b2_ref[...]


def _matmuls(x_pad, wt, wp, b2):
    blk = NP // 8
    return pl.pallas_call(
        _mm_body,
        grid=(8,),
        in_specs=[
            pl.BlockSpec((blk, D), lambda i: (i, 0)),
            pl.BlockSpec((D, D), lambda i: (0, 0)),
            pl.BlockSpec((D, D), lambda i: (0, 0)),
            pl.BlockSpec((1, D), lambda i: (0, 0)),
        ],
        out_specs=[
            pl.BlockSpec((blk, D), lambda i: (i, 0)),
            pl.BlockSpec((blk, D), lambda i: (i, 0)),
        ],
        out_shape=[
            jax.ShapeDtypeStruct((NP, D), jnp.float32),
            jax.ShapeDtypeStruct((NP, D), jnp.float32),
        ],
    )(x_pad, wt, wp, b2)


def _sc_body(xt, xp, xpad, srcv, dstv, dvec, out,
             m, rows, dstc, srcc, dc, sbuf, lbuf, dbuf, xc, pc, acc,
             off_ref, sem):
    wid = lax.axis_index("s") * 2 + lax.axis_index("c")
    lo = wid * R
    cnt = jnp.minimum(R, N - lo)
    hi = lo + cnt

    neg = jnp.full((16,), NEG, jnp.float32)
    zf = jnp.zeros((16,), jnp.float32)
    zi = jnp.zeros((16,), jnp.int32)

    def _init_m(i, _):
        m[pl.ds(i * 16, 16)] = neg
        return 0
    lax.fori_loop(0, R * D // 16, _init_m, 0)
    for i in range(CAP // 16):
        sbuf[pl.ds(i * 16, 16)] = zi
    for j in range(D // 16):
        acc[pl.ds(j * 16, 16)] = zf
    off_ref[0] = 0

    def flush(count):
        pltpu.async_copy(xt.at[sbuf], rows, sem).wait()

        def ebody(i, _):
            dl = lbuf[pl.ds(i, 16)][0]
            dvs = dbuf[pl.ds(i, 16)][0]
            base = dl * D
            for j in range(D // 16):
                sl = pl.ds(base + j * 16, 16)
                m[sl] = jnp.maximum(m[sl], dvs * rows[i, pl.ds(j * 16, 16)])
            return 0
        lax.fori_loop(0, count, ebody, 0)

    def chunk_body(c, _):
        ebase = c * CHUNK
        pltpu.sync_copy(dstv.at[pl.ds(ebase, CHUNK)], dstc)
        pltpu.sync_copy(srcv.at[pl.ds(ebase, CHUNK)], srcc)
        pltpu.sync_copy(dvec.at[pl.ds(ebase, CHUNK)], dc)

        lanes = lax.iota(jnp.int32, 16)

        def gbody(g, _):
            gs = pl.ds(g * 16, 16)
            dsts = dstc[gs]
            srcs = srcc[gs]
            dvs = dc[gs]
            mask = (dsts >= lo) & (dsts < hi)
            mi = jnp.where(mask, 1, 0)
            s = mi
            for k in (1, 2, 4, 8):
                s = s + _dg(s, lanes ^ k)
            npop = s[0]
            off0 = off_ref[0]

            @pl.when(off0 > FLUSH_AT)
            def _():
                flush(off0)
                off_ref[0] = 0

            off = off_ref[0]
            dstl = dsts - lo

            def ebody(t, mic):
                tv = jnp.where(mic > 0, lanes, 16)
                for k in (1, 2, 4, 8):
                    tv = jnp.minimum(tv, _dg(tv, lanes ^ k))
                pos = off + t
                sbuf[pl.ds(pos, 16)] = _dg(srcs, tv)
                lbuf[pl.ds(pos, 16)] = _dg(dstl, tv)
                dbuf[pl.ds(pos, 16)] = _dg(dvs, tv)
                return jnp.where(lanes == tv, 0, mic)
            lax.fori_loop(0, npop, ebody, mi)
            off_ref[0] = off + npop
            return 0
        lax.fori_loop(0, NG, gbody, 0)
        return 0
    lax.fori_loop(0, NCHUNKS, chunk_body, 0)

    offd = off_ref[0]

    @pl.when(offd > 0)
    def _():
        flush(offd)

    def fbody(c2, _):
        base = lo + c2 * 16
        pltpu.sync_copy(xpad.at[pl.ds(base, 16)], xc)
        pltpu.sync_copy(xp.at[pl.ds(base, 16)], pc)

        def rbody(r, _):
            n = c2 * 16 + r

            @pl.when(n < cnt)
            def _():
                mb = n * D
                for j in range(D // 16):
                    jl = pl.ds(j * 16, 16)
                    mv = m[pl.ds(mb + j * 16, 16)]
                    hv = jnp.where(mv > NEG, mv + pc[r, jl], xc[r, jl])
                    acc[jl] = acc[jl] + hv
            return 0
        lax.fori_loop(0, 16, rbody, 0)
        return 0
    lax.fori_loop(0, R // 16, fbody, 0)

    pltpu.sync_copy(acc, out.at[wid])


_sc_kernel = functools.partial(
    pl.kernel,
    mesh=plsc.VectorSubcoreMesh(core_axis_name="c", subcore_axis_name="s"),
    out_type=jax.ShapeDtypeStruct((NW, D), jnp.float32),
    scratch_types=[
        pltpu.VMEM((R * D,), jnp.float32),
        pltpu.VMEM((CAP, D), jnp.float32),
        pltpu.VMEM((CHUNK,), jnp.int32),
        pltpu.VMEM((CHUNK,), jnp.int32),
        pltpu.VMEM((CHUNK,), jnp.float32),
        pltpu.VMEM((CAP,), jnp.int32),
        pltpu.VMEM((CAP + 16,), jnp.int32),
        pltpu.VMEM((CAP + 16,), jnp.float32),
        pltpu.VMEM((16, D), jnp.float32),
        pltpu.VMEM((16, D), jnp.float32),
        pltpu.VMEM((D,), jnp.float32),
        pltpu.SMEM((1,), jnp.int32),
        pltpu.SemaphoreType.DMA,
    ],
)(_sc_body)


def kernel(x, d, edge_index, W_theta, b_theta, W_phi, b_phi):
    src = edge_index[0]
    dst = edge_index[1]
    x_pad = jnp.zeros((NP, D), jnp.float32).at[:N].set(x)
    b2 = (b_theta + b_phi).reshape(1, D)
    XT, XPb = _matmuls(x_pad, W_theta, W_phi, b2)
    partials = _sc_kernel(XT, XPb, x_pad, src, dst, d)
    return jnp.sum(partials, axis=0, keepdims=True) * (1.0 / N)
